# baseline (speedup 1.0000x reference)
import functools

import jax
import jax.numpy as jnp
from jax import lax
from jax.experimental import pallas as pl
from jax.experimental.pallas import tpu as pltpu
from jax.experimental.pallas import tpu_sc as plsc

_VOCAB = 1000000
_SEQ = 200
_DIM = 64
_BATCH = 4096
_SCALE = 8.0

_NC, _NS = 2, 16
_NW = _NC * _NS
_TOTAL = _BATCH * _SEQ
_PER_W = _TOTAL // _NW
_CHUNK = 512
_SUB = 128
_KSUB = _CHUNK // _SUB
_ROUNDS = _PER_W // _CHUNK
_LANES = 16


def _fire_round(idx_hbm, table_hbm, idx_v, rows_v, sem, wid, g):
    row0 = wid * (_PER_W // _SUB) + g * _KSUB
    pltpu.sync_copy(idx_hbm.at[pl.ds(row0, _KSUB)], idx_v)
    for j in range(_KSUB):
        pltpu.async_copy(
            table_hbm.at[idx_v.at[j]],
            rows_v.at[pl.ds(j * _SUB, _SUB)],
            sem,
        )


def _drain_round(table_hbm, idx_v, rows_v, sem):
    for j in range(_KSUB):
        pltpu.make_async_copy(
            table_hbm.at[idx_v.at[j]],
            rows_v.at[pl.ds(j * _SUB, _SUB)],
            sem,
        ).wait()


def _compute_round(rows_v, pos_v, g):
    gofs = g * _CHUNK

    @pl.loop(0, _CHUNK, unroll=4)
    def _row(r):
        pr = lax.rem(gofs + r, _SEQ)
        for k in range(_DIM // _LANES):
            sl = pl.ds(k * _LANES, _LANES)
            rows_v[r, sl] = rows_v[r, sl] * _SCALE + pos_v[pr, sl]


def _body(idx_hbm, table_hbm, pos_hbm, out_hbm,
          pos_v, idx_v0, idx_v1, rows_v0, rows_v1, gsem0, gsem1):
    wid = lax.axis_index("s") * _NC + lax.axis_index("c")
    base = wid * _PER_W

    pltpu.sync_copy(pos_hbm, pos_v)

    idx_v = (idx_v0, idx_v1)
    rows_v = (rows_v0, rows_v1)
    gsem = (gsem0, gsem1)

    _fire_round(idx_hbm, table_hbm, idx_v[0], rows_v[0], gsem[0], wid, 0)

    def _round(g, b, fire_next):
        if fire_next:
            _fire_round(idx_hbm, table_hbm, idx_v[1 - b], rows_v[1 - b],
                        gsem[1 - b], wid, g + 1)
        _drain_round(table_hbm, idx_v[b], rows_v[b], gsem[b])
        _compute_round(rows_v[b], pos_v, g)
        pltpu.sync_copy(rows_v[b], out_hbm.at[pl.ds(base + g * _CHUNK, _CHUNK)])

    @pl.loop(0, _ROUNDS - 2, step=2)
    def _steady(g0):
        for b in range(2):
            _round(g0 + b, b, fire_next=True)

    _round(_ROUNDS - 2, (_ROUNDS - 2) % 2, fire_next=True)
    _round(_ROUNDS - 1, (_ROUNDS - 1) % 2, fire_next=False)


@jax.jit
def _embed(idx2d, token_table, pos_table):
    mesh = plsc.VectorSubcoreMesh(
        core_axis_name="c", subcore_axis_name="s",
        num_cores=_NC, num_subcores=_NS,
    )
    kern = pl.kernel(
        _body,
        out_type=jax.ShapeDtypeStruct((_TOTAL, _DIM), jnp.float32),
        mesh=mesh,
        compiler_params=pltpu.CompilerParams(use_tc_tiling_on_sc=False),
        scratch_types=[
            pltpu.VMEM((_SEQ, _DIM), jnp.float32),
            pltpu.VMEM((_KSUB, _SUB), jnp.int32),
            pltpu.VMEM((_KSUB, _SUB), jnp.int32),
            pltpu.VMEM((_CHUNK, _DIM), jnp.float32),
            pltpu.VMEM((_CHUNK, _DIM), jnp.float32),
            pltpu.SemaphoreType.DMA,
            pltpu.SemaphoreType.DMA,
        ],
    )
    return kern(idx2d, token_table, pos_table)


def kernel(inputs, token_table, pos_table):
    idx2d = inputs.reshape(_TOTAL // _SUB, _SUB).astype(jnp.int32)
    out = _embed(idx2d, token_table, pos_table)
    return out.reshape(_BATCH, _SEQ, _DIM)


# speedup vs baseline: 1.2534x; 1.2534x over previous
import jax
import jax.numpy as jnp
from jax import lax
from jax.experimental import pallas as pl
from jax.experimental.pallas import tpu as pltpu
from jax.experimental.pallas import tpu_sc as plsc

_VOCAB = 1000000
_SEQ = 200
_DIM = 64
_BATCH = 4096
_SCALE = 8.0

_NC, _NS = 2, 16
_NW = _NC * _NS
_TOTAL = _BATCH * _SEQ
_PER_W = _TOTAL // _NW
_SEQ_PER_CHUNK = 2
_CHUNK = _SEQ * _SEQ_PER_CHUNK
_SUB = 100
_KSUB = _CHUNK // _SUB
_ROUNDS = _PER_W // _CHUNK
_LANES = 16


def _fire_round(idx_hbm, table_hbm, idx_v, rows_v, sem, wid, g):
    row0 = wid * (_PER_W // _SUB) + g * _KSUB
    pltpu.sync_copy(idx_hbm.at[pl.ds(row0, _KSUB)], idx_v)
    for j in range(_KSUB):
        pltpu.async_copy(
            table_hbm.at[idx_v.at[j]],
            rows_v.at[pl.ds(j * _SUB, _SUB)],
            sem,
        )


def _drain_round(table_hbm, idx_v, rows_v, sem):
    for j in range(_KSUB):
        pltpu.make_async_copy(
            table_hbm.at[idx_v.at[j]],
            rows_v.at[pl.ds(j * _SUB, _SUB)],
            sem,
        ).wait()


def _compute_round(rows_v, pos_v):
    for s in range(_SEQ_PER_CHUNK):
        @plsc.parallel_loop(0, _SEQ, unroll=8)
        def _row(r):
            rr = s * _SEQ + r
            for k in range(_DIM // _LANES):
                sl = pl.ds(k * _LANES, _LANES)
                rows_v[rr, sl] = rows_v[rr, sl] * _SCALE + pos_v[r, sl]


def _body(idx_hbm, table_hbm, pos_hbm, out_hbm,
          pos_v, idx_v0, idx_v1, rows_v0, rows_v1, gsem0, gsem1):
    wid = lax.axis_index("s") * _NC + lax.axis_index("c")
    base = wid * _PER_W

    pltpu.sync_copy(pos_hbm, pos_v)

    idx_v = (idx_v0, idx_v1)
    rows_v = (rows_v0, rows_v1)
    gsem = (gsem0, gsem1)

    _fire_round(idx_hbm, table_hbm, idx_v[0], rows_v[0], gsem[0], wid, 0)

    def _round(g, b, fire_next):
        if fire_next:
            _fire_round(idx_hbm, table_hbm, idx_v[1 - b], rows_v[1 - b],
                        gsem[1 - b], wid, g + 1)
        _drain_round(table_hbm, idx_v[b], rows_v[b], gsem[b])
        _compute_round(rows_v[b], pos_v)
        pltpu.sync_copy(rows_v[b], out_hbm.at[pl.ds(base + g * _CHUNK, _CHUNK)])

    @pl.loop(0, _ROUNDS - 2, step=2)
    def _steady(g0):
        for b in range(2):
            _round(g0 + b, b, fire_next=True)

    _round(_ROUNDS - 2, (_ROUNDS - 2) % 2, fire_next=True)
    _round(_ROUNDS - 1, (_ROUNDS - 1) % 2, fire_next=False)


@jax.jit
def _embed(idx2d, token_table, pos_table):
    mesh = plsc.VectorSubcoreMesh(
        core_axis_name="c", subcore_axis_name="s",
        num_cores=_NC, num_subcores=_NS,
    )
    kern = pl.kernel(
        _body,
        out_type=jax.ShapeDtypeStruct((_TOTAL, _DIM), jnp.float32),
        mesh=mesh,
        compiler_params=pltpu.CompilerParams(use_tc_tiling_on_sc=False),
        scratch_types=[
            pltpu.VMEM((_SEQ, _DIM), jnp.float32),
            pltpu.VMEM((_KSUB, _SUB), jnp.int32),
            pltpu.VMEM((_KSUB, _SUB), jnp.int32),
            pltpu.VMEM((_CHUNK, _DIM), jnp.float32),
            pltpu.VMEM((_CHUNK, _DIM), jnp.float32),
            pltpu.SemaphoreType.DMA,
            pltpu.SemaphoreType.DMA,
        ],
    )
    return kern(idx2d, token_table, pos_table)


def kernel(inputs, token_table, pos_table):
    idx2d = inputs.reshape(_TOTAL // _SUB, _SUB).astype(jnp.int32)
    out = _embed(idx2d, token_table, pos_table)
    return out.reshape(_BATCH, _SEQ, _DIM)
